# Initial kernel scaffold; baseline (speedup 1.0000x reference)
import jax
import jax.numpy as jnp
from jax.experimental import pallas as pl


def kernel(indices, table):
    raise NotImplementedError("write your pallas kernel here")


# speedup vs baseline: 4.5816x; 4.5816x over previous
import functools

import jax
import jax.numpy as jnp
from jax import lax
from jax.experimental import pallas as pl
from jax.experimental.pallas import tpu as pltpu
from jax.experimental.pallas import tpu_sc as plsc

B, L, D = 4096, 50, 64
TOTAL = B * L
NC, NS = 2, 16
NW = NC * NS
ROWS_PER_W = TOTAL // NW
CHUNK = 128
NCHUNK = ROWS_PER_W // CHUNK
NBUF = 5
NGROUP = NCHUNK // NBUF


@functools.partial(
    pl.kernel,
    out_type=jax.ShapeDtypeStruct((TOTAL, D), jnp.float32),
    mesh=plsc.VectorSubcoreMesh(core_axis_name="c", subcore_axis_name="s"),
    compiler_params=pltpu.CompilerParams(use_tc_tiling_on_sc=False),
    scratch_types=(
        [pltpu.VMEM((NCHUNK, CHUNK), jnp.int32)]
        + [pltpu.VMEM((CHUNK, D), jnp.float32) for _ in range(NBUF)]
        + [pltpu.SemaphoreType.DMA for _ in range(NBUF)]
    ),
)
def _sc_gather(idx_hbm, table_hbm, out_hbm, idx_v, *rest):
    bufs = rest[:NBUF]
    sems = rest[NBUF:]
    wid = lax.axis_index("s") * NC + lax.axis_index("c")
    base = wid * ROWS_PER_W

    pltpu.sync_copy(idx_hbm.at[wid], idx_v)

    def group(g, _):
        j0 = g * NBUF
        handles = [
            pltpu.async_copy(table_hbm.at[idx_v.at[j0 + b]], bufs[b], sems[b])
            for b in range(NBUF)
        ]
        for b in range(NBUF):
            handles[b].wait()
            pltpu.sync_copy(
                bufs[b], out_hbm.at[pl.ds(base + (j0 + b) * CHUNK, CHUNK)]
            )
        return _

    lax.fori_loop(0, NGROUP, group, None)


def kernel(indices, table):
    idx = indices.reshape(NW, NCHUNK, CHUNK).astype(jnp.int32)
    out = _sc_gather(idx, table)
    return out.reshape(B, L, D)


# speedup vs baseline: 4.6527x; 1.0155x over previous
import functools

import jax
import jax.numpy as jnp
from jax import lax
from jax.experimental import pallas as pl
from jax.experimental.pallas import tpu as pltpu
from jax.experimental.pallas import tpu_sc as plsc

B, L, D = 4096, 50, 64
TOTAL = B * L
NC, NS = 2, 16
NW = NC * NS
ROWS_PER_W = TOTAL // NW
CHUNK = 128
NCHUNK = ROWS_PER_W // CHUNK
NBUF = 5
NBANK = 2
BANK_ROWS = NBUF * CHUNK
NPAIR = NCHUNK // (NBUF * NBANK)


@functools.partial(
    pl.kernel,
    out_type=jax.ShapeDtypeStruct((TOTAL, D), jnp.float32),
    mesh=plsc.VectorSubcoreMesh(core_axis_name="c", subcore_axis_name="s"),
    compiler_params=pltpu.CompilerParams(use_tc_tiling_on_sc=False),
    scratch_types=(
        [pltpu.VMEM((NCHUNK, CHUNK), jnp.int32)]
        + [pltpu.VMEM((BANK_ROWS, D), jnp.float32) for _ in range(NBANK)]
        + [pltpu.SemaphoreType.DMA for _ in range(NBANK * NBUF)]
        + [pltpu.SemaphoreType.DMA for _ in range(NBANK)]
    ),
)
def _sc_gather(idx_hbm, table_hbm, out_hbm, idx_v, *rest):
    banks = rest[:NBANK]
    gsems = [rest[NBANK + k * NBUF : NBANK + (k + 1) * NBUF] for k in range(NBANK)]
    wsems = rest[NBANK + NBANK * NBUF :]
    wid = lax.axis_index("s") * NC + lax.axis_index("c")
    base = wid * ROWS_PER_W

    pltpu.sync_copy(idx_hbm.at[wid], idx_v)

    def pair(p, _):
        handles = []
        for k in range(NBANK):
            @pl.when(p > 0)
            def _drain(k=k):
                pltpu.make_async_copy(
                    banks[k], out_hbm.at[pl.ds(base, BANK_ROWS)], wsems[k]
                ).wait()

            handles.append([
                pltpu.async_copy(
                    table_hbm.at[idx_v.at[p * NBANK * NBUF + k * NBUF + b]],
                    banks[k].at[pl.ds(b * CHUNK, CHUNK)],
                    gsems[k][b],
                )
                for b in range(NBUF)
            ])
        for k in range(NBANK):
            for b in range(NBUF):
                handles[k][b].wait()
            row0 = base + (p * NBANK + k) * BANK_ROWS
            pltpu.async_copy(
                banks[k], out_hbm.at[pl.ds(row0, BANK_ROWS)], wsems[k]
            )
        return _

    lax.fori_loop(0, NPAIR, pair, None)
    for k in range(NBANK):
        pltpu.make_async_copy(
            banks[k], out_hbm.at[pl.ds(base, BANK_ROWS)], wsems[k]
        ).wait()


def kernel(indices, table):
    idx = indices.reshape(NW, NCHUNK, CHUNK).astype(jnp.int32)
    out = _sc_gather(idx, table)
    return out.reshape(B, L, D)


# speedup vs baseline: 4.6663x; 1.0029x over previous
import functools

import jax
import jax.numpy as jnp
from jax import lax
from jax.experimental import pallas as pl
from jax.experimental.pallas import tpu as pltpu
from jax.experimental.pallas import tpu_sc as plsc

B, L, D = 4096, 50, 64
NC, NS = 2, 16
NW = NC * NS
B_PER_W = B // NW
NBUF = 8
NBANK = 2
NPAIR = B_PER_W // (NBUF * NBANK)


@functools.partial(
    pl.kernel,
    out_type=jax.ShapeDtypeStruct((B, L, D), jnp.float32),
    mesh=plsc.VectorSubcoreMesh(core_axis_name="c", subcore_axis_name="s"),
    compiler_params=pltpu.CompilerParams(use_tc_tiling_on_sc=False),
    scratch_types=(
        [pltpu.VMEM((B_PER_W, L), jnp.int32)]
        + [pltpu.VMEM((NBUF, L, D), jnp.float32) for _ in range(NBANK)]
        + [pltpu.SemaphoreType.DMA for _ in range(NBANK * NBUF)]
        + [pltpu.SemaphoreType.DMA for _ in range(NBANK)]
    ),
)
def _sc_gather(idx_hbm, table_hbm, out_hbm, idx_v, *rest):
    banks = rest[:NBANK]
    gsems = [rest[NBANK + k * NBUF : NBANK + (k + 1) * NBUF] for k in range(NBANK)]
    wsems = rest[NBANK + NBANK * NBUF :]
    wid = lax.axis_index("s") * NC + lax.axis_index("c")
    base = wid * B_PER_W

    pltpu.sync_copy(idx_hbm.at[pl.ds(base, B_PER_W)], idx_v)

    def pair(p, _):
        handles = []
        for k in range(NBANK):
            @pl.when(p > 0)
            def _drain(k=k):
                pltpu.make_async_copy(
                    banks[k], out_hbm.at[pl.ds(base, NBUF)], wsems[k]
                ).wait()

            handles.append([
                pltpu.async_copy(
                    table_hbm.at[idx_v.at[p * NBANK * NBUF + k * NBUF + b]],
                    banks[k].at[b],
                    gsems[k][b],
                )
                for b in range(NBUF)
            ])
        for k in range(NBANK):
            for b in range(NBUF):
                handles[k][b].wait()
            row0 = base + (p * NBANK + k) * NBUF
            pltpu.async_copy(banks[k], out_hbm.at[pl.ds(row0, NBUF)], wsems[k])
        return _

    lax.fori_loop(0, NPAIR, pair, None)
    for k in range(NBANK):
        pltpu.make_async_copy(
            banks[k], out_hbm.at[pl.ds(base, NBUF)], wsems[k]
        ).wait()


def kernel(indices, table):
    return _sc_gather(indices.astype(jnp.int32), table)


# speedup vs baseline: 4.9598x; 1.0629x over previous
import functools

import jax
import jax.numpy as jnp
from jax import lax
from jax.experimental import pallas as pl
from jax.experimental.pallas import tpu as pltpu
from jax.experimental.pallas import tpu_sc as plsc

B, L, D = 4096, 50, 64
V = 100000
NC, NS = 2, 16
NW = NC * NS
DPW = D // NW
LANES = 16
NVEC = B // LANES
UNROLL = 8


@functools.partial(
    pl.kernel,
    out_type=jax.ShapeDtypeStruct((L, D, B), jnp.float32),
    mesh=plsc.VectorSubcoreMesh(core_axis_name="c", subcore_axis_name="s"),
    compiler_params=pltpu.CompilerParams(
        use_tc_tiling_on_sc=False, needs_layout_passes=False
    ),
    scratch_types=(
        [pltpu.VMEM((V,), jnp.float32)]
        + [pltpu.VMEM((B,), jnp.int32) for _ in range(2)]
        + [pltpu.VMEM((B,), jnp.float32) for _ in range(2)]
        + [pltpu.VMEM_SHARED((L, B), jnp.int32)]
        + [pltpu.SemaphoreType.DMA for _ in range(5)]
    ),
)
def _sc_lookup(idx_hbm, tab_hbm, out_hbm, row_v, ib0, ib1, ob0, ob1, sidx,
               rsem, is0, is1, os0, os1):
    ibufs, isems = (ib0, ib1), (is0, is1)
    obufs, osems = (ob0, ob1), (os0, os1)
    core = lax.axis_index("c")
    sid = lax.axis_index("s")
    wid = sid * NC + core

    @pl.when(sid == 0)
    def _stage_idx():
        pltpu.sync_copy(idx_hbm, sidx)

    d_first = wid * DPW
    row_h = pltpu.async_copy(tab_hbm.at[d_first], row_v, rsem)
    plsc.subcore_barrier()

    for k in range(DPW):
        d = d_first + k
        if k == 0:
            row_h.wait()
        else:
            pltpu.sync_copy(tab_hbm.at[d], row_v)

        pltpu.async_copy(sidx.at[0], ibufs[0], isems[0])
        pltpu.async_copy(sidx.at[1], ibufs[1], isems[1])

        def pair(p, _):
            for t in range(2):
                l = 2 * p + t
                ib, ob = ibufs[t], obufs[t]
                pltpu.make_async_copy(sidx.at[0], ib, isems[t]).wait()
                @pl.when(p > 0)
                def _drain_out(t=t, ob=ob):
                    pltpu.make_async_copy(
                        ob, out_hbm.at[0, 0], osems[t]
                    ).wait()

                def gather(i, _, ib=ib, ob=ob):
                    base = i * (LANES * UNROLL)
                    for u in range(UNROLL):
                        off = base + u * LANES
                        iv = ib[pl.ds(off, LANES)]
                        ob[pl.ds(off, LANES)] = plsc.load_gather(row_v, [iv])
                    return _

                lax.fori_loop(0, NVEC // UNROLL, gather, None)
                pltpu.async_copy(ob, out_hbm.at[l, d], osems[t])
                @pl.when(l + 2 < L)
                def _prefetch(l=l, ib=ib, t=t):
                    pltpu.async_copy(sidx.at[l + 2], ib, isems[t])
            return _

        lax.fori_loop(0, L // 2, pair, None)
        for t in range(2):
            pltpu.make_async_copy(obufs[t], out_hbm.at[0, 0], osems[t]).wait()


def kernel(indices, table):
    idx_t = indices.T.astype(jnp.int32)
    tab_t = table.T
    out_t = _sc_lookup(idx_t, tab_t)
    return jnp.transpose(out_t, (2, 0, 1))


# speedup vs baseline: 7.2841x; 1.4686x over previous
import functools

import jax
import jax.numpy as jnp
from jax import lax
from jax.experimental import pallas as pl
from jax.experimental.pallas import tpu as pltpu
from jax.experimental.pallas import tpu_sc as plsc

B, L, D = 4096, 50, 64
V = 100000
NC, NS = 2, 16
NW = NC * NS
DPW = D // NW
LANES = 16
NVEC = B // LANES
UNROLL = 8


@functools.partial(
    pl.kernel,
    out_type=jax.ShapeDtypeStruct((L, D, B), jnp.float32),
    mesh=plsc.VectorSubcoreMesh(core_axis_name="c", subcore_axis_name="s"),
    compiler_params=pltpu.CompilerParams(
        use_tc_tiling_on_sc=False, needs_layout_passes=False
    ),
    scratch_types=(
        [pltpu.VMEM((V,), jnp.float32)]
        + [pltpu.VMEM((B,), jnp.int32) for _ in range(2)]
        + [pltpu.VMEM((B,), jnp.float32) for _ in range(2)]
        + [pltpu.VMEM_SHARED((L, B), jnp.int32)]
        + [pltpu.SemaphoreType.DMA for _ in range(5)]
    ),
)
def _sc_lookup(idx_hbm, tab_hbm, out_hbm, row_v, ib0, ib1, ob0, ob1, sidx,
               rsem, is0, is1, os0, os1):
    ibufs, isems = (ib0, ib1), (is0, is1)
    obufs, osems = (ob0, ob1), (os0, os1)
    core = lax.axis_index("c")
    sid = lax.axis_index("s")
    wid = sid * NC + core

    @pl.when(sid == 0)
    def _stage_idx():
        pltpu.sync_copy(idx_hbm, sidx)

    d_first = wid * DPW
    row_h = pltpu.async_copy(tab_hbm.at[d_first], row_v, rsem)
    plsc.subcore_barrier()

    for k in range(DPW):
        d = d_first + k
        if k == 0:
            row_h.wait()
        else:
            pltpu.sync_copy(tab_hbm.at[d], row_v)

        pltpu.async_copy(sidx.at[0], ibufs[0], isems[0])
        pltpu.async_copy(sidx.at[1], ibufs[1], isems[1])

        def pair(p, _):
            for t in range(2):
                l = 2 * p + t
                ib, ob = ibufs[t], obufs[t]
                pltpu.make_async_copy(sidx.at[0], ib, isems[t]).wait()
                @pl.when(p > 0)
                def _drain_out(t=t, ob=ob):
                    pltpu.make_async_copy(
                        ob, out_hbm.at[0, 0], osems[t]
                    ).wait()

                @plsc.parallel_loop(0, B, step=LANES, unroll=UNROLL)
                def _gather(off, ib=ib, ob=ob):
                    iv = ib[pl.ds(off, LANES)]
                    ob[pl.ds(off, LANES)] = plsc.load_gather(row_v, [iv])
                pltpu.async_copy(ob, out_hbm.at[l, d], osems[t])
                @pl.when(l + 2 < L)
                def _prefetch(l=l, ib=ib, t=t):
                    pltpu.async_copy(sidx.at[l + 2], ib, isems[t])
            return _

        lax.fori_loop(0, L // 2, pair, None)
        for t in range(2):
            pltpu.make_async_copy(obufs[t], out_hbm.at[0, 0], osems[t]).wait()


def kernel(indices, table):
    idx_t = indices.T.astype(jnp.int32)
    tab_t = table.T
    out_t = _sc_lookup(idx_t, tab_t)
    return jnp.transpose(out_t, (2, 0, 1))


# speedup vs baseline: 10.9343x; 1.5011x over previous
import functools

import jax
import jax.numpy as jnp
from jax import lax
from jax.experimental import pallas as pl
from jax.experimental.pallas import tpu as pltpu
from jax.experimental.pallas import tpu_sc as plsc

B, L, D = 4096, 50, 64
V = 100000
NC, NS = 2, 16
NW = NC * NS
DPW = D // NW
LANES = 16
NVEC = B // LANES
UNROLL = 8


@functools.partial(
    pl.kernel,
    out_type=jax.ShapeDtypeStruct((L, D // 8, B // 128, 8, 128), jnp.float32),
    mesh=plsc.VectorSubcoreMesh(core_axis_name="c", subcore_axis_name="s"),
    compiler_params=pltpu.CompilerParams(
        use_tc_tiling_on_sc=False, needs_layout_passes=False
    ),
    scratch_types=(
        [pltpu.VMEM((V,), jnp.float32)]
        + [pltpu.VMEM((B,), jnp.int32) for _ in range(2)]
        + [pltpu.VMEM((B // 128, 128), jnp.float32) for _ in range(2)]
        + [pltpu.VMEM_SHARED((L, B), jnp.int32)]
        + [pltpu.SemaphoreType.DMA for _ in range(5)]
    ),
)
def _sc_lookup(idx_hbm, tab_hbm, out_hbm, row_v, ib0, ib1, ob0, ob1, sidx,
               rsem, is0, is1, os0, os1):
    ibufs, isems = (ib0, ib1), (is0, is1)
    obufs, osems = (ob0, ob1), (os0, os1)
    core = lax.axis_index("c")
    sid = lax.axis_index("s")
    wid = sid * NC + core

    @pl.when(sid == 0)
    def _stage_idx():
        pltpu.sync_copy(idx_hbm, sidx)

    d_first = wid * DPW
    row_h = pltpu.async_copy(tab_hbm.at[d_first], row_v, rsem)
    plsc.subcore_barrier()

    for k in range(DPW):
        d = d_first + k
        dr, dsub = d // 8, d % 8
        if k == 0:
            row_h.wait()
        else:
            pltpu.sync_copy(tab_hbm.at[d], row_v)

        pltpu.async_copy(sidx.at[0], ibufs[0], isems[0])
        pltpu.async_copy(sidx.at[1], ibufs[1], isems[1])

        def pair(p, _):
            for t in range(2):
                l = 2 * p + t
                ib, ob = ibufs[t], obufs[t]
                pltpu.make_async_copy(sidx.at[0], ib, isems[t]).wait()
                @pl.when(p > 0)
                def _drain_out(t=t, ob=ob):
                    pltpu.make_async_copy(
                        ob, out_hbm.at[0, 0, :, 0, :], osems[t]
                    ).wait()

                @plsc.parallel_loop(0, B, step=LANES, unroll=UNROLL)
                def _gather(off, ib=ib, ob=ob):
                    iv = ib[pl.ds(off, LANES)]
                    ob[off // 128, pl.ds(off % 128, LANES)] = (
                        plsc.load_gather(row_v, [iv])
                    )
                pltpu.async_copy(
                    ob, out_hbm.at[l, dr, :, dsub, :], osems[t]
                )
                @pl.when(l + 2 < L)
                def _prefetch(l=l, ib=ib, t=t):
                    pltpu.async_copy(sidx.at[l + 2], ib, isems[t])
            return _

        lax.fori_loop(0, L // 2, pair, None)
        for t in range(2):
            pltpu.make_async_copy(
                obufs[t], out_hbm.at[0, 0, :, 0, :], osems[t]
            ).wait()


def kernel(indices, table):
    idx_t = indices.T.astype(jnp.int32)
    tab_t = table.T
    out5 = _sc_lookup(idx_t, tab_t)
    return out5.transpose(2, 4, 0, 1, 3).reshape(B, L, D)


# speedup vs baseline: 10.9924x; 1.0053x over previous
import functools

import jax
import jax.numpy as jnp
from jax import lax
from jax.experimental import pallas as pl
from jax.experimental.pallas import tpu as pltpu
from jax.experimental.pallas import tpu_sc as plsc

B, L, D = 4096, 50, 64
V = 100000
NC, NS = 2, 16
NW = NC * NS
DPW = D // NW
LANES = 16
NVEC = B // LANES
UNROLL = 16


@functools.partial(
    pl.kernel,
    out_type=jax.ShapeDtypeStruct((L, D // 8, B // 128, 8, 128), jnp.float32),
    mesh=plsc.VectorSubcoreMesh(core_axis_name="c", subcore_axis_name="s"),
    compiler_params=pltpu.CompilerParams(
        use_tc_tiling_on_sc=False, needs_layout_passes=False
    ),
    scratch_types=(
        [pltpu.VMEM((V,), jnp.float32)]
        + [pltpu.VMEM((B,), jnp.int32) for _ in range(2)]
        + [pltpu.VMEM((B // 128, 128), jnp.float32) for _ in range(2)]
        + [pltpu.VMEM_SHARED((L, B), jnp.int32)]
        + [pltpu.SemaphoreType.DMA for _ in range(5)]
    ),
)
def _sc_lookup(idx_hbm, tab_hbm, out_hbm, row_v, ib0, ib1, ob0, ob1, sidx,
               rsem, is0, is1, os0, os1):
    ibufs, isems = (ib0, ib1), (is0, is1)
    obufs, osems = (ob0, ob1), (os0, os1)
    core = lax.axis_index("c")
    sid = lax.axis_index("s")
    wid = sid * NC + core

    @pl.when(sid == 0)
    def _stage_idx():
        pltpu.sync_copy(idx_hbm, sidx)

    d_first = wid * DPW
    row_h = pltpu.async_copy(tab_hbm.at[d_first], row_v, rsem)
    plsc.subcore_barrier()

    for k in range(DPW):
        d = d_first + k
        dr, dsub = d // 8, d % 8
        if k == 0:
            row_h.wait()
        else:
            pltpu.sync_copy(tab_hbm.at[d], row_v)

        pltpu.async_copy(sidx.at[0], ibufs[0], isems[0])
        pltpu.async_copy(sidx.at[1], ibufs[1], isems[1])

        def pair(p, _):
            for t in range(2):
                l = 2 * p + t
                ib, ob = ibufs[t], obufs[t]
                pltpu.make_async_copy(sidx.at[0], ib, isems[t]).wait()
                @pl.when(p > 0)
                def _drain_out(t=t, ob=ob):
                    pltpu.make_async_copy(
                        ob, out_hbm.at[0, 0, :, 0, :], osems[t]
                    ).wait()

                @plsc.parallel_loop(0, B, step=LANES, unroll=UNROLL)
                def _gather(off, ib=ib, ob=ob):
                    iv = ib[pl.ds(off, LANES)]
                    ob[off // 128, pl.ds(off % 128, LANES)] = (
                        plsc.load_gather(row_v, [iv])
                    )
                pltpu.async_copy(
                    ob, out_hbm.at[l, dr, :, dsub, :], osems[t]
                )
                @pl.when(l + 2 < L)
                def _prefetch(l=l, ib=ib, t=t):
                    pltpu.async_copy(sidx.at[l + 2], ib, isems[t])
            return _

        lax.fori_loop(0, L // 2, pair, None)
        for t in range(2):
            pltpu.make_async_copy(
                obufs[t], out_hbm.at[0, 0, :, 0, :], osems[t]
            ).wait()


def kernel(indices, table):
    idx_t = indices.T.astype(jnp.int32)
    tab_t = table.T
    out5 = _sc_lookup(idx_t, tab_t)
    return out5.transpose(2, 4, 0, 1, 3).reshape(B, L, D)
